# Initial kernel scaffold; baseline (speedup 1.0000x reference)
import jax
import jax.numpy as jnp
from jax.experimental import pallas as pl


def kernel(atom_features, bond_features, three_body_basis, bond_atom_indices, triple_bond_indices, W_update, b_update, W_fusion, b_fusion):
    raise NotImplementedError("write your pallas kernel here")


# speedup vs baseline: 10.1344x; 10.1344x over previous
import functools

import jax
import jax.numpy as jnp
from jax import lax
from jax.experimental import pallas as pl
from jax.experimental.pallas import tpu as pltpu
from jax.experimental.pallas import tpu_sc as plsc

NA = 10000
NAP = 10240
NB = 320000
NT = 1280000
DB = 64
DF = 128

NCORES = 2
NSUB = 16
CHUNK_B = 16000
NCHUNK = NB // CHUNK_B
PASSES = NCHUNK // NCORES
TPT = NT // NSUB
TB = 2000
NBLK = TPT // TB
G = 128
STAGE_CAP = 2176
ROWS_PT = CHUNK_B // NSUB
ZROWS = 128


def _tc_proj(atom_features, W_update, b_update):
    BR = 1024

    def body(a_ref, w_ref, b_ref, o_ref):
        o_ref[...] = (
            jnp.dot(a_ref[...], w_ref[...], preferred_element_type=jnp.float32)
            + b_ref[0:1, :]
        )

    return pl.pallas_call(
        body,
        grid=(NAP // BR,),
        in_specs=[
            pl.BlockSpec((BR, DF), lambda i: (i, 0)),
            pl.BlockSpec((DF, DB), lambda i: (0, 0)),
            pl.BlockSpec((8, DB), lambda i: (0, 0)),
        ],
        out_specs=pl.BlockSpec((BR, DB), lambda i: (i, 0)),
        out_shape=jax.ShapeDtypeStruct((NAP, DB), jnp.float32),
    )(atom_features, W_update, jnp.broadcast_to(b_update, (8, DB)))


def _tc_fusion(summed, bond_features, W_fusion, b_fusion):
    BR = 4000

    def body(s_ref, bf_ref, w_ref, b_ref, o_ref):
        o_ref[...] = (
            bf_ref[...]
            + jnp.dot(s_ref[...], w_ref[...], preferred_element_type=jnp.float32)
            + b_ref[0:1, :]
        )

    return pl.pallas_call(
        body,
        grid=(NB // BR,),
        in_specs=[
            pl.BlockSpec((BR, DB), lambda i: (i, 0)),
            pl.BlockSpec((BR, DF), lambda i: (i, 0)),
            pl.BlockSpec((DB, DF), lambda i: (0, 0)),
            pl.BlockSpec((8, DF), lambda i: (0, 0)),
        ],
        out_specs=pl.BlockSpec((BR, DF), lambda i: (i, 0)),
        out_shape=jax.ShapeDtypeStruct((NB, DF), jnp.float32),
    )(summed, bond_features, W_fusion, jnp.broadcast_to(b_fusion, (8, DF)))


def _sc_middle(proj, bond2, tb0, tb1, basis):
    mesh = plsc.VectorSubcoreMesh(core_axis_name="c", subcore_axis_name="s")

    @functools.partial(
        pl.kernel,
        out_type=jax.ShapeDtypeStruct((NB, DB), jnp.float32),
        mesh=mesh,
        compiler_params=pltpu.CompilerParams(
            needs_layout_passes=False, use_tc_tiling_on_sc=False),
        scratch_types=[
            pltpu.VMEM_SHARED((NB,), jnp.int32),
            pltpu.VMEM_SHARED((CHUNK_B + 8, DB), jnp.float32),
            pltpu.VMEM((TB,), jnp.int32),
            pltpu.VMEM((TB,), jnp.int32),
            pltpu.VMEM((STAGE_CAP,), jnp.int32),
            pltpu.VMEM((STAGE_CAP,), jnp.int32),
            pltpu.VMEM((STAGE_CAP,), jnp.int32),
            pltpu.VMEM((G,), jnp.int32),
            pltpu.VMEM((G,), jnp.int32),
            pltpu.VMEM((G,), jnp.int32),
            pltpu.VMEM((G,), jnp.int32),
            pltpu.VMEM((G, DB), jnp.float32),
            pltpu.VMEM((G, DB), jnp.float32),
            pltpu.VMEM((ZROWS, DB), jnp.float32),
            pltpu.SemaphoreType.DMA,
            pltpu.SemaphoreType.DMA,
            pltpu.SemaphoreType.DMA,
        ],
    )
    def k(proj_hbm, bond2_hbm, tb0_hbm, tb1_hbm, basis_hbm, out_hbm,
          bond2_sh, acc, tb0_blk, tb1_blk, st_tid, st_t1, st_lc,
          tidbuf, t1buf, lcbuf, thirdbuf, brows, prows, zbuf,
          semb, semp, semt):
        cid = lax.axis_index("c")
        sid = lax.axis_index("s")
        iota16 = lax.iota(jnp.int32, 16)

        for p in range(NB // NSUB // TB):
            b0 = sid * (NB // NSUB) + p * TB
            pltpu.sync_copy(bond2_hbm.at[pl.ds(b0, TB)], tb0_blk)
            pltpu.sync_copy(tb0_blk, bond2_sh.at[pl.ds(b0, TB)])

        def zfill(r, _):
            for c4 in range(DB // 16):
                zbuf[r, pl.ds(c4 * 16, 16)] = jnp.zeros((16,), jnp.float32)
            return 0

        lax.fori_loop(0, ZROWS, zfill, 0)
        plsc.subcore_barrier()

        def flush(start, cnt):
            for v in range(G // 16):
                off = start + v * 16
                valid = (off + iota16) < cnt
                lc = st_lc[pl.ds(off, 16)]
                t1 = st_t1[pl.ds(off, 16)]
                ti = st_tid[pl.ds(off, 16)]
                lcbuf[pl.ds(v * 16, 16)] = jnp.where(valid, lc, CHUNK_B)
                t1buf[pl.ds(v * 16, 16)] = jnp.where(valid, t1, 0)
                tidbuf[pl.ds(v * 16, 16)] = jnp.where(valid, ti, 0)
            cpb = pltpu.async_copy(basis_hbm.at[tidbuf], brows, semb)
            pltpu.async_copy(bond2_sh.at[t1buf], thirdbuf, semt).wait()
            pltpu.async_copy(proj_hbm.at[thirdbuf], prows, semp).wait()
            cpb.wait()

            def mul(r, _):
                for c4 in range(DB // 16):
                    s_ = pl.ds(c4 * 16, 16)
                    brows[r, s_] = brows[r, s_] * prows[r, s_]
                return 0

            lax.fori_loop(0, G, mul, 0)
            pltpu.sync_copy(brows, acc.at[lcbuf], add=True)

        def do_chunk(kk, _):
            chunk = kk * NCORES + cid
            lo = chunk * CHUNK_B
            for q in range(ROWS_PT // 125):
                pltpu.sync_copy(zbuf.at[pl.ds(0, 125)],
                                acc.at[pl.ds(sid * ROWS_PT + q * 125, 125)])
            plsc.subcore_barrier()

            def do_block(b, cnt):
                t0 = sid * TPT + b * TB
                pltpu.sync_copy(tb0_hbm.at[pl.ds(t0, TB)], tb0_blk)
                pltpu.sync_copy(tb1_hbm.at[pl.ds(t0, TB)], tb1_blk)

                def compact(i, cnt):
                    off = i * 16
                    rel = tb0_blk[pl.ds(off, 16)] - lo
                    m = (rel >= 0) & (rel < CHUNK_B)
                    t1 = tb1_blk[pl.ds(off, 16)]
                    tid = (t0 + off) + iota16
                    plsc.store_compressed(st_lc.at[pl.ds(cnt, 16)], rel, mask=m)
                    plsc.store_compressed(st_t1.at[pl.ds(cnt, 16)], t1, mask=m)
                    plsc.store_compressed(st_tid.at[pl.ds(cnt, 16)], tid, mask=m)
                    return cnt + jnp.sum(m.astype(jnp.int32))

                cnt = lax.fori_loop(0, TB // 16, compact, cnt)
                nf = cnt // G

                def fl(g, _):
                    flush(g * G, cnt)
                    return 0

                lax.fori_loop(0, nf, fl, 0)

                @pl.when(nf > 0)
                def _():
                    for v in range(G // 16):
                        src = pl.ds(nf * G + v * 16, 16)
                        dst = pl.ds(v * 16, 16)
                        st_lc[dst] = st_lc[src]
                        st_t1[dst] = st_t1[src]
                        st_tid[dst] = st_tid[src]

                return cnt - nf * G

            cnt = lax.fori_loop(0, NBLK, do_block, jnp.int32(0))

            @pl.when(cnt > 0)
            def _():
                flush(0, cnt)

            plsc.subcore_barrier()
            for q in range(ROWS_PT // 125):
                r0 = sid * ROWS_PT + q * 125
                pltpu.sync_copy(acc.at[pl.ds(r0, 125)], brows.at[pl.ds(0, 125)])
                pltpu.sync_copy(brows.at[pl.ds(0, 125)], out_hbm.at[pl.ds(lo + r0, 125)])
            plsc.subcore_barrier()
            return 0

        lax.fori_loop(0, PASSES, do_chunk, 0)

    return k(proj, bond2, tb0, tb1, basis)


def kernel(atom_features, bond_features, three_body_basis, bond_atom_indices,
           triple_bond_indices, W_update, b_update, W_fusion, b_fusion):
    atoms_p = jnp.pad(atom_features, ((0, NAP - NA), (0, 0)))
    proj = _tc_proj(atoms_p, W_update, b_update)
    bond2 = bond_atom_indices[:, 1]
    tb0 = triple_bond_indices[:, 0]
    tb1 = triple_bond_indices[:, 1]
    summed = _sc_middle(proj, bond2, tb0, tb1, three_body_basis)
    return _tc_fusion(summed, bond_features, W_fusion, b_fusion)
